# baseline (speedup 1.0000x reference)
import functools

import jax
import jax.numpy as jnp
import numpy as np
from jax.experimental import pallas as pl
from jax.experimental.pallas import tpu as pltpu

_ROT0 = (13, 15, 26, 6)
_ROT1 = (17, 29, 16, 24)


def _rotl(x, r):
    return jax.lax.shift_left(x, np.uint32(r)) | jax.lax.shift_right_logical(
        x, np.uint32(32 - r))


def _threefry_bits(i):
    k0 = np.uint32(0)
    k1 = np.uint32(42)
    k2 = k0 ^ k1 ^ np.uint32(0x1BD11BDA)
    ks = (k0, k1, k2)
    x0 = jnp.zeros_like(i) + ks[0]
    x1 = i + ks[1]
    for d in range(5):
        for r in (_ROT0 if d % 2 == 0 else _ROT1):
            x0 = x0 + x1
            x1 = _rotl(x1, r) ^ x0
        x0 = x0 + ks[(d + 1) % 3]
        x1 = x1 + ks[(d + 2) % 3] + np.uint32(d + 1)
    return x0 ^ x1


def _body(p_ref, o_ref, bv_ref, bi_ref, *, vocab, chunk):
    j = pl.program_id(0)
    nsteps = pl.num_programs(0)

    @pl.when(j == 0)
    def _init():
        bv_ref[:] = jnp.full_like(bv_ref, -jnp.inf)
        bi_ref[:] = jnp.zeros_like(bi_ref)

    p = p_ref[:]
    rows = jax.lax.broadcasted_iota(jnp.uint32, p.shape, 0)
    cols = jax.lax.broadcasted_iota(jnp.uint32, p.shape, 1)
    cols = cols + (j * chunk).astype(jnp.uint32)
    lin = rows * np.uint32(vocab) + cols

    bits = _threefry_bits(lin)
    fb = jax.lax.shift_right_logical(bits, np.uint32(9)) | np.uint32(0x3F800000)
    f = jax.lax.bitcast_convert_type(fb, jnp.float32) - np.float32(1.0)
    u = jnp.maximum(np.float32(1e-20), f + np.float32(1e-20))
    g = -jnp.log(-jnp.log(u))
    logp = jnp.log(jnp.maximum(p, np.float32(1e-20)))
    v = logp + g

    valid = cols < np.uint32(vocab)
    v = jnp.where(valid, v, -jnp.inf)

    m = jnp.max(v, axis=1, keepdims=True)
    colsi = cols.astype(jnp.int32)
    idx = jnp.min(jnp.where(v == m, colsi, np.int32(2**31 - 1)), axis=1,
                  keepdims=True)

    upd = m > bv_ref[:]
    bi_ref[:] = jnp.where(upd, idx, bi_ref[:])
    bv_ref[:] = jnp.where(upd, m, bv_ref[:])

    @pl.when(j == nsteps - 1)
    def _done():
        o_ref[:] = bi_ref[:]


def kernel(probs):
    b, vocab = probs.shape
    chunk = 2048
    nsteps = (vocab + chunk - 1) // chunk
    out = pl.pallas_call(
        functools.partial(_body, vocab=vocab, chunk=chunk),
        grid=(nsteps,),
        in_specs=[pl.BlockSpec((b, chunk), lambda j: (0, j))],
        out_specs=pl.BlockSpec((b, 1), lambda j: (0, 0)),
        out_shape=jax.ShapeDtypeStruct((b, 1), jnp.int32),
        scratch_shapes=[
            pltpu.VMEM((b, 1), jnp.float32),
            pltpu.VMEM((b, 1), jnp.int32),
        ],
    )(probs)
    return out.reshape(b)
